# baseline (speedup 1.0000x reference)
import functools

import jax
import jax.numpy as jnp
from jax import lax
from jax.experimental import pallas as pl
from jax.experimental.pallas import tpu as pltpu
from jax.experimental.pallas import tpu_sc as plsc

_VOCAB = 1000000
_DIM = 16
_BATCH = 16384

_NC = 2
_NS = 16
_NW = _NC * _NS
_BPW = _BATCH // _NW
_NBUF = 16

_mesh = plsc.VectorSubcoreMesh(core_axis_name="c", subcore_axis_name="s")


@functools.partial(
    pl.kernel,
    mesh=_mesh,
    out_type=jax.ShapeDtypeStruct((_BATCH, 128), jnp.float32),
    scratch_types=[
        pltpu.VMEM((_BPW + 16,), jnp.int32),
        pltpu.VMEM((_NBUF, _DIM, 128), jnp.float32),
        pltpu.VMEM((_BPW, 128), jnp.float32),
        [pltpu.SemaphoreType.DMA] * _NBUF,
    ],
    compiler_params=pltpu.CompilerParams(needs_layout_passes=False),
)
def _gather_kernel(zt_hbm, idx_hbm, out_hbm, idx_v, win_v, rows_v, sems):
    wid = lax.axis_index("s") * _NC + lax.axis_index("c")
    base = wid * _BPW
    pltpu.sync_copy(idx_hbm.at[pl.ds(base, _BPW)], idx_v.at[pl.ds(0, _BPW)])

    row_ids = lax.iota(jnp.int32, 16)

    def fire(j, slot):
        v = idx_v[pl.ds(j, 16)][0]
        col = pl.multiple_of((v // 128) * 128, 128)
        pltpu.async_copy(
            zt_hbm.at[:, pl.ds(col, 128)],
            win_v.at[slot],
            sems[slot],
        )

    for b in range(_NBUF):
        fire(b, b)

    def group(g, _):
        for b in range(_NBUF):
            j = g * _NBUF + b
            pltpu.make_async_copy(
                zt_hbm.at[:, pl.ds(0, 128)], win_v.at[b], sems[b]
            ).wait()
            v = idx_v[pl.ds(j, 16)][0]
            vl = lax.rem(v, 128)
            row = plsc.load_gather(
                win_v.at[b], [row_ids, jnp.full((16,), vl, jnp.int32)]
            )
            rows_v[j, pl.ds(0, _DIM)] = row

            @pl.when(j + _NBUF < _BPW)
            def _():
                fire(j + _NBUF, b)

        return ()

    lax.fori_loop(0, _BPW // _NBUF, group, ())
    pltpu.sync_copy(rows_v, out_hbm.at[pl.ds(base, _BPW), :])


def kernel(Z, indices):
    idx = indices.astype(jnp.int32)
    blob = _gather_kernel(Z.T, idx)
    return blob[:, :_DIM]


# speedup vs baseline: 1.1024x; 1.1024x over previous
import functools

import jax
import jax.numpy as jnp
from jax import lax
from jax.experimental import pallas as pl
from jax.experimental.pallas import tpu as pltpu
from jax.experimental.pallas import tpu_sc as plsc

_VOCAB = 1000000
_DIM = 16
_BATCH = 16384

_NC = 2
_NS = 16
_NW = _NC * _NS
_BPW = _BATCH // _NW
_NBUF = 8

_mesh = plsc.VectorSubcoreMesh(core_axis_name="c", subcore_axis_name="s")


@functools.partial(
    pl.kernel,
    mesh=_mesh,
    out_type=jax.ShapeDtypeStruct((_DIM, _BATCH), jnp.float32),
    scratch_types=[
        pltpu.VMEM((_BPW + 16,), jnp.int32),
        pltpu.VMEM((_NBUF, _DIM, 128), jnp.float32),
        pltpu.VMEM((_BPW, _DIM), jnp.float32),
        pltpu.VMEM((_DIM, _BPW), jnp.float32),
        [pltpu.SemaphoreType.DMA] * _NBUF,
    ],
    compiler_params=pltpu.CompilerParams(needs_layout_passes=False),
)
def _gather_kernel(zt_hbm, idx_hbm, out_hbm, idx_v, win_v, rows_v, blk_v, sems):
    wid = lax.axis_index("s") * _NC + lax.axis_index("c")
    base = wid * _BPW
    pltpu.sync_copy(idx_hbm.at[pl.ds(base, _BPW)], idx_v.at[pl.ds(0, _BPW)])

    row_ids = lax.iota(jnp.int32, 16)

    def fire(j, slot):
        v = idx_v[pl.ds(j, 16)][0]
        col = pl.multiple_of((v // 128) * 128, 128)
        pltpu.async_copy(
            zt_hbm.at[:, pl.ds(col, 128)],
            win_v.at[slot],
            sems[slot],
        )

    for b in range(_NBUF):
        fire(b, b)

    def group(g, _):
        for b in range(_NBUF):
            j = g * _NBUF + b
            pltpu.make_async_copy(
                zt_hbm.at[:, pl.ds(0, 128)], win_v.at[b], sems[b]
            ).wait()
            v = idx_v[pl.ds(j, 16)][0]
            vl = lax.rem(v, 128)
            row = plsc.load_gather(
                win_v.at[b], [row_ids, jnp.full((16,), vl, jnp.int32)]
            )
            rows_v[j, :] = row

            @pl.when(j + _NBUF < _BPW)
            def _():
                fire(j + _NBUF, b)

        return ()

    lax.fori_loop(0, _BPW // _NBUF, group, ())

    for e in range(_DIM):
        col_ids = jnp.full((16,), e, jnp.int32)
        for c in range(_BPW // 16):
            vals = plsc.load_gather(rows_v, [c * 16 + row_ids, col_ids])
            blk_v[e, pl.ds(c * 16, 16)] = vals
    pltpu.sync_copy(blk_v, out_hbm.at[:, pl.ds(base, _BPW)])


def kernel(Z, indices):
    idx = indices.astype(jnp.int32)
    out_t = _gather_kernel(Z.T, idx)
    return out_t.T


# speedup vs baseline: 1.1054x; 1.0028x over previous
import functools

import jax
import jax.numpy as jnp
from jax import lax
from jax.experimental import pallas as pl
from jax.experimental.pallas import tpu as pltpu
from jax.experimental.pallas import tpu_sc as plsc

_VOCAB = 1000000
_DIM = 16
_BATCH = 16384

_NC = 2
_NS = 16
_NW = _NC * _NS
_BPW = _BATCH // _NW
_NBUF = 8

_mesh = plsc.VectorSubcoreMesh(core_axis_name="c", subcore_axis_name="s")


@functools.partial(
    pl.kernel,
    mesh=_mesh,
    out_type=jax.ShapeDtypeStruct((_DIM, _BATCH), jnp.float32),
    scratch_types=[
        pltpu.VMEM((_BPW + 16,), jnp.int32),
        pltpu.VMEM((_NBUF, _DIM, 128), jnp.float32),
        pltpu.VMEM((_BPW, _DIM), jnp.float32),
        pltpu.VMEM((_DIM, _BPW), jnp.float32),
        [pltpu.SemaphoreType.DMA] * _NBUF,
    ],
    compiler_params=pltpu.CompilerParams(needs_layout_passes=False),
)
def _gather_kernel(zt_hbm, idx_hbm, out_hbm, idx_v, win_v, rows_v, blk_v, sems):
    wid = lax.axis_index("s") * _NC + lax.axis_index("c")
    base = wid * _BPW
    pltpu.sync_copy(idx_hbm.at[pl.ds(base, _BPW)], idx_v.at[pl.ds(0, _BPW)])

    row_ids = lax.iota(jnp.int32, 16)

    def fire(j, slot):
        v = idx_v[pl.ds(j, 16)][0]
        col = pl.multiple_of((v // 128) * 128, 128)
        pltpu.async_copy(
            zt_hbm.at[pl.ds(0, 8), pl.ds(col, 128)],
            win_v.at[slot, pl.ds(0, 8)],
            sems[slot],
        )
        pltpu.async_copy(
            zt_hbm.at[pl.ds(8, 8), pl.ds(col, 128)],
            win_v.at[slot, pl.ds(8, 8)],
            sems[slot],
        )

    for b in range(_NBUF):
        fire(b, b)

    def group(g, _):
        for b in range(_NBUF):
            j = g * _NBUF + b
            pltpu.make_async_copy(
                zt_hbm.at[:, pl.ds(0, 128)], win_v.at[b], sems[b]
            ).wait()
            v = idx_v[pl.ds(j, 16)][0]
            vl = lax.rem(v, 128)
            row = plsc.load_gather(
                win_v.at[b], [row_ids, jnp.full((16,), vl, jnp.int32)]
            )
            rows_v[j, :] = row

            @pl.when(j + _NBUF < _BPW)
            def _():
                fire(j + _NBUF, b)

        return ()

    lax.fori_loop(0, _BPW // _NBUF, group, ())

    for e in range(_DIM):
        col_ids = jnp.full((16,), e, jnp.int32)
        for c in range(_BPW // 16):
            vals = plsc.load_gather(rows_v, [c * 16 + row_ids, col_ids])
            blk_v[e, pl.ds(c * 16, 16)] = vals
    pltpu.sync_copy(blk_v, out_hbm.at[:, pl.ds(base, _BPW)])


def kernel(Z, indices):
    idx = indices.astype(jnp.int32)
    out_t = _gather_kernel(Z.T, idx)
    return out_t.T
